# baseline (speedup 1.0000x reference)
import functools

import jax
import jax.numpy as jnp
from jax import lax
from jax.experimental import pallas as pl
from jax.experimental.pallas import tpu as pltpu
from jax.experimental.pallas import tpu_sc as plsc

B, N, D, D3 = 1024, 200, 128, 384
_NC, _NS, _L = 2, 16, 16
NW = _NC * _NS
BPW = B // NW
KDEPTH = 8


def _sc_body(idx_c_hbm, idx_a_hbm, idx_t_hbm,
             emb_c_hbm, emb_a_hbm, emb_t_hbm, out_hbm,
             idx_c_v, idx_a_v, idx_t_v, rows_c, rows_a, rows_t, cat_v,
             gsem, osem):
    wid = lax.axis_index("s") * _NC + lax.axis_index("c")
    base = wid * BPW

    pltpu.sync_copy(idx_c_hbm.at[pl.ds(base, BPW)], idx_c_v)
    pltpu.sync_copy(idx_a_hbm.at[pl.ds(base, BPW)], idx_a_v)
    pltpu.sync_copy(idx_t_hbm.at[pl.ds(base, BPW)], idx_t_v)

    c_c = pltpu.async_copy(emb_c_hbm.at[idx_c_v], rows_c, gsem)
    c_a = pltpu.async_copy(emb_a_hbm.at[idx_a_v], rows_a, gsem)
    c_t = pltpu.async_copy(emb_t_hbm.at[idx_t_v], rows_t, gsem)
    c_c.wait()
    c_a.wait()
    c_t.wait()

    for s in range(BPW):
        for k3, src in enumerate((rows_c, rows_a, rows_t)):
            for q in range(D // _L):
                cat_v[s, 0, pl.ds(k3 * D + q * _L, _L)] = src[s, pl.ds(q * _L, _L)]

    def _dst(n):
        return out_hbm.at[pl.ds(base, BPW), pl.ds(n, 1), :]

    for n in range(KDEPTH):
        pltpu.async_copy(cat_v, _dst(n), osem)

    def _body(g, carry):
        pltpu.async_copy(cat_v, _dst(g + KDEPTH), osem)
        pltpu.make_async_copy(cat_v, _dst(0), osem).wait()
        return carry

    lax.fori_loop(0, N - KDEPTH, _body, 0)
    for _ in range(KDEPTH):
        pltpu.make_async_copy(cat_v, _dst(0), osem).wait()


def kernel(x_t, idx_C, idx_A, idx_T, emb_C, emb_A, emb_T):
    mesh = plsc.VectorSubcoreMesh(core_axis_name="c", subcore_axis_name="s",
                                  num_cores=_NC, num_subcores=_NS)
    run = functools.partial(
        pl.kernel,
        mesh=mesh,
        out_type=jax.ShapeDtypeStruct((B, N, D3), jnp.float32),
        scratch_types=[
            pltpu.VMEM((BPW,), jnp.int32),
            pltpu.VMEM((BPW,), jnp.int32),
            pltpu.VMEM((BPW,), jnp.int32),
            pltpu.VMEM((BPW, D), jnp.float32),
            pltpu.VMEM((BPW, D), jnp.float32),
            pltpu.VMEM((BPW, D), jnp.float32),
            pltpu.VMEM((BPW, 1, D3), jnp.float32),
            pltpu.SemaphoreType.DMA,
            pltpu.SemaphoreType.DMA,
        ],
    )(_sc_body)
    return run(idx_C.astype(jnp.int32), idx_A.astype(jnp.int32),
               idx_T.astype(jnp.int32), emb_C, emb_A, emb_T)


# speedup vs baseline: 1.0458x; 1.0458x over previous
import functools

import jax
import jax.numpy as jnp
from jax import lax
from jax.experimental import pallas as pl
from jax.experimental.pallas import tpu as pltpu
from jax.experimental.pallas import tpu_sc as plsc

B, N, D, D3 = 1024, 200, 128, 384
_NC, _NS, _L = 2, 16, 16
NW = _NC * _NS
BPW = B // NW
BB = 16


def _sc_gather_body(idx_c_hbm, idx_a_hbm, idx_t_hbm,
                    emb_c_hbm, emb_a_hbm, emb_t_hbm, fe_hbm,
                    idx_c_v, idx_a_v, idx_t_v, rows_c, rows_a, rows_t,
                    gsem, osem):
    wid = lax.axis_index("s") * _NC + lax.axis_index("c")
    base = wid * BPW

    pltpu.sync_copy(idx_c_hbm.at[pl.ds(base, BPW)], idx_c_v)
    pltpu.sync_copy(idx_a_hbm.at[pl.ds(base, BPW)], idx_a_v)
    pltpu.sync_copy(idx_t_hbm.at[pl.ds(base, BPW)], idx_t_v)

    c_c = pltpu.async_copy(emb_c_hbm.at[idx_c_v], rows_c, gsem)
    c_a = pltpu.async_copy(emb_a_hbm.at[idx_a_v], rows_a, gsem)
    c_t = pltpu.async_copy(emb_t_hbm.at[idx_t_v], rows_t, gsem)
    c_c.wait()
    c_a.wait()
    c_t.wait()

    w_c = pltpu.async_copy(rows_c, fe_hbm.at[pl.ds(base, BPW), pl.ds(0, D)], osem)
    w_a = pltpu.async_copy(rows_a, fe_hbm.at[pl.ds(base, BPW), pl.ds(D, D)], osem)
    w_t = pltpu.async_copy(rows_t, fe_hbm.at[pl.ds(base, BPW), pl.ds(2 * D, D)], osem)
    w_c.wait()
    w_a.wait()
    w_t.wait()


def _sc_gather(idx_C, idx_A, idx_T, emb_C, emb_A, emb_T):
    mesh = plsc.VectorSubcoreMesh(core_axis_name="c", subcore_axis_name="s",
                                  num_cores=_NC, num_subcores=_NS)
    run = functools.partial(
        pl.kernel,
        mesh=mesh,
        out_type=jax.ShapeDtypeStruct((B, D3), jnp.float32),
        scratch_types=[
            pltpu.VMEM((BPW,), jnp.int32),
            pltpu.VMEM((BPW,), jnp.int32),
            pltpu.VMEM((BPW,), jnp.int32),
            pltpu.VMEM((BPW, D), jnp.float32),
            pltpu.VMEM((BPW, D), jnp.float32),
            pltpu.VMEM((BPW, D), jnp.float32),
            pltpu.SemaphoreType.DMA,
            pltpu.SemaphoreType.DMA,
        ],
    )(_sc_gather_body)
    return run(idx_C, idx_A, idx_T, emb_C, emb_A, emb_T)


def _bcast_kernel(fe_ref, out_ref):
    out_ref[...] = jnp.broadcast_to(fe_ref[:, :, :], (BB, N, D3))


def _tc_broadcast(fold_emb):
    return pl.pallas_call(
        _bcast_kernel,
        grid=(B // BB,),
        in_specs=[pl.BlockSpec((BB, 1, D3), lambda i: (i, 0, 0))],
        out_specs=pl.BlockSpec((BB, N, D3), lambda i: (i, 0, 0)),
        out_shape=jax.ShapeDtypeStruct((B, N, D3), jnp.float32),
    )(fold_emb)


def kernel(x_t, idx_C, idx_A, idx_T, emb_C, emb_A, emb_T):
    fold_emb = _sc_gather(idx_C.astype(jnp.int32), idx_A.astype(jnp.int32),
                          idx_T.astype(jnp.int32), emb_C, emb_A, emb_T)
    return _tc_broadcast(fold_emb.reshape(B, 1, D3))
